# baseline (speedup 1.0000x reference)
import functools

import jax
import jax.numpy as jnp
from jax import lax
from jax.experimental import pallas as pl
from jax.experimental.pallas import tpu as pltpu
from jax.experimental.pallas import tpu_sc as plsc

N = 10000
D = 128
NC = 2
NS = 16
LN = 16
NW = NC * NS
C = 128
NP = 10240
RPT = NP // NS
ZB = 64


def _sc_mesh():
    return plsc.VectorSubcoreMesh(
        core_axis_name="c", subcore_axis_name="s", num_cores=NC, num_subcores=NS
    )


CD = 1024


def _make_deg(e_pad):
    per_tile = e_pad // NW
    nchunks = per_tile // CD

    def body(dst_hbm, w_hbm, out_hbm, dbuf, wbuf, acc):
        c = lax.axis_index("c")
        s = lax.axis_index("s")
        wid = c * NS + s

        zero = jnp.zeros((LN,), jnp.float32)

        def zloop(i, _):
            acc[pl.ds(i * LN, LN)] = zero
            return 0

        lax.fori_loop(0, N // LN, zloop, 0)

        base = wid * per_tile

        def chunk(k, _):
            off = base + k * CD
            pltpu.sync_copy(dst_hbm.at[pl.ds(off, CD)], dbuf)
            pltpu.sync_copy(w_hbm.at[pl.ds(off, CD)], wbuf)

            def group(t, _):
                iv = dbuf[pl.ds(t * LN, LN)]
                wv = wbuf[pl.ds(t * LN, LN)]
                plsc.addupdate_scatter(acc, [iv], wv)
                return 0

            lax.fori_loop(0, CD // LN, group, 0)
            return 0

        lax.fori_loop(0, nchunks, chunk, 0)
        pltpu.sync_copy(acc, out_hbm.at[wid])

    return pl.kernel(
        body,
        out_type=jax.ShapeDtypeStruct((NW, N), jnp.float32),
        mesh=_sc_mesh(),
        compiler_params=pltpu.CompilerParams(needs_layout_passes=False),
        scratch_types=[
            pltpu.VMEM((CD,), jnp.int32),
            pltpu.VMEM((CD,), jnp.float32),
            pltpu.VMEM((N,), jnp.float32),
        ],
    )


def _make_spmm(e_pad):
    nchunks = e_pad // C
    totq = nchunks // (4 * NS)
    assert nchunks % (4 * NS) == 0 and totq >= 2

    def body(h_hbm, src_hbm, dst_hbm, w_hbm, out_hbm,
             s0, s1, s2, s3, d0, d1, d2, d3, w0, w1, w2, w3,
             r0, r1, zbuf, acc,
             si0, si1, si2, si3, sg0, sg1, ss0, ss1):
        c = lax.axis_index("c")
        s = lax.axis_index("s")

        @pl.when(c == 0)
        def _work():
            _spmm_tile(h_hbm, src_hbm, dst_hbm, w_hbm, out_hbm,
                       s0, s1, s2, s3, d0, d1, d2, d3, w0, w1, w2, w3,
                       r0, r1, zbuf, acc,
                       si0, si1, si2, si3, sg0, sg1, ss0, ss1, s, totq)

    def _spmm_tile(h_hbm, src_hbm, dst_hbm, w_hbm, out_hbm,
                   s0, s1, s2, s3, d0, d1, d2, d3, w0, w1, w2, w3,
                   r0, r1, zbuf, acc,
                   si0, si1, si2, si3, sg0, sg1, ss0, ss1, s, nq):
        zero = jnp.zeros((LN,), jnp.float32)

        def zloop(i, _):
            zbuf[i // (D // LN), pl.ds((i % (D // LN)) * LN, LN)] = zero
            return 0

        lax.fori_loop(0, ZB * (D // LN), zloop, 0)

        rbase = s * RPT

        def zcopy(i, _):
            pltpu.sync_copy(zbuf, acc.at[pl.ds(rbase + i * ZB, ZB)])
            return 0

        lax.fori_loop(0, RPT // ZB, zcopy, 0)
        plsc.subcore_barrier()

        base = s * (4 * nq) * C
        cmax = 4 * nq - 1
        SB = [s0, s1, s2, s3]
        DB = [d0, d1, d2, d3]
        WB = [w0, w1, w2, w3]
        SI = [si0, si1, si2, si3]

        def idx_start(ck, j):
            off = base + jnp.minimum(ck, cmax) * C
            return (pltpu.async_copy(src_hbm.at[pl.ds(off, C)], SB[j], SI[j]),
                    pltpu.async_copy(dst_hbm.at[pl.ds(off, C)], DB[j], SI[j]),
                    pltpu.async_copy(w_hbm.at[pl.ds(off, C)], WB[j], SI[j]))

        def idx_wait(j):
            pltpu.make_async_copy(src_hbm.at[pl.ds(base, C)], SB[j], SI[j]).wait()
            pltpu.make_async_copy(dst_hbm.at[pl.ds(base, C)], DB[j], SI[j]).wait()
            pltpu.make_async_copy(w_hbm.at[pl.ds(base, C)], WB[j], SI[j]).wait()

        def gather_start(j, r, sem):
            return pltpu.async_copy(h_hbm.at[SB[j]], r, sem)

        def gather_wait(j, r, sem):
            pltpu.make_async_copy(h_hbm.at[SB[j]], r, sem).wait()

        def scale(r, wref):
            def sc16(t, _):
                wv = wref[pl.ds(t * LN, LN)]
                for l in range(LN):
                    wsv = jnp.full((LN,), wv[l], jnp.float32)
                    e = t * LN + l
                    for j in range(D // LN):
                        r[e, pl.ds(j * LN, LN)] = r[e, pl.ds(j * LN, LN)] * wsv
                return 0

            lax.fori_loop(0, C // LN, sc16, 0)

        pro = [idx_start(j, j) for j in range(4)]
        for t in pro[0]:
            t.wait()
        gather_start(0, r0, sg0)
        for t in pro[1]:
            t.wait()
        gather_start(1, r1, sg1)

        def quad(q, _):
            c0 = 4 * q
            gather_wait(0, r0, sg0)
            scale(r0, w0)
            sc_a = pltpu.async_copy(r0, acc.at[d0], ss0, add=True)
            gather_wait(1, r1, sg1)
            scale(r1, w1)
            sc_b = pltpu.async_copy(r1, acc.at[d1], ss1, add=True)
            sc_a.wait()
            idx_wait(2)
            gather_start(2, r0, sg0)
            idx_start(c0 + 4, 0)
            sc_b.wait()
            idx_wait(3)
            gather_start(3, r1, sg1)
            idx_start(c0 + 5, 1)
            gather_wait(2, r0, sg0)
            scale(r0, w2)
            sc_c = pltpu.async_copy(r0, acc.at[d2], ss0, add=True)
            gather_wait(3, r1, sg1)
            scale(r1, w3)
            sc_d = pltpu.async_copy(r1, acc.at[d3], ss1, add=True)
            sc_c.wait()
            idx_wait(0)
            gather_start(0, r0, sg0)
            idx_start(c0 + 6, 2)
            sc_d.wait()
            idx_wait(1)
            gather_start(1, r1, sg1)
            idx_start(c0 + 7, 3)
            return 0

        lax.fori_loop(0, nq, quad, 0)

        gather_wait(0, r0, sg0)
        gather_wait(1, r1, sg1)
        idx_wait(2)
        idx_wait(3)

        plsc.subcore_barrier()

        def wback(i, _):
            pltpu.sync_copy(acc.at[pl.ds(rbase + i * ZB, ZB)],
                            out_hbm.at[pl.ds(rbase + i * ZB, ZB)])
            return 0

        lax.fori_loop(0, RPT // ZB, wback, 0)

    return pl.kernel(
        body,
        out_type=jax.ShapeDtypeStruct((NP, D), jnp.float32),
        mesh=_sc_mesh(),
        compiler_params=pltpu.CompilerParams(needs_layout_passes=False),
        scratch_types=(
            [pltpu.VMEM((C,), jnp.int32)] * 8
            + [pltpu.VMEM((C,), jnp.float32)] * 4
            + [pltpu.VMEM((C, D), jnp.float32)] * 2
            + [pltpu.VMEM((ZB, D), jnp.float32),
               pltpu.VMEM_SHARED((NP, D), jnp.float32)]
            + [pltpu.SemaphoreType.DMA] * 8
        ),
    )


def _dis_body(p1_ref, p2_ref, o1_ref, o2_ref):
    o1_ref[...] = lax.rsqrt(1.0 + jnp.sum(p1_ref[...], axis=0, keepdims=True))
    o2_ref[...] = lax.rsqrt(1.0 + jnp.sum(p2_ref[...], axis=0, keepdims=True))


def _dis_call(p1, p2):
    return pl.pallas_call(
        _dis_body,
        out_shape=(jax.ShapeDtypeStruct((1, N), jnp.float32),
                   jax.ShapeDtypeStruct((1, N), jnp.float32)),
    )(p1, p2)


BM = 1000


def _mm_body(x_ref, w_ref, dis_ref, o_ref):
    h = jnp.dot(x_ref[...], w_ref[...], preferred_element_type=jnp.float32)
    o_ref[...] = h * dis_ref[...]


def _mm_call(x, w, dis_col):
    return pl.pallas_call(
        _mm_body,
        grid=(N // BM,),
        in_specs=[
            pl.BlockSpec((BM, D), lambda i: (i, 0)),
            pl.BlockSpec((D, D), lambda i: (0, 0)),
            pl.BlockSpec((BM, 1), lambda i: (i, 0)),
        ],
        out_specs=pl.BlockSpec((BM, D), lambda i: (i, 0)),
        out_shape=jax.ShapeDtypeStruct((N, D), jnp.float32),
    )(x, w, dis_col)


def _finmm_body(sa_ref, hp_ref, dis_ref, b_ref, w_ref, o_ref):
    h = dis_ref[...] * (sa_ref[...] + hp_ref[...]) + b_ref[...]
    o_ref[...] = dis_ref[...] * jnp.dot(h, w_ref[...],
                                        preferred_element_type=jnp.float32)


def _finmm_call(sa, hp, dis_col, b_row, w):
    return pl.pallas_call(
        _finmm_body,
        grid=(N // BM,),
        in_specs=[
            pl.BlockSpec((BM, D), lambda i: (i, 0)),
            pl.BlockSpec((BM, D), lambda i: (i, 0)),
            pl.BlockSpec((BM, 1), lambda i: (i, 0)),
            pl.BlockSpec((1, D), lambda i: (0, 0)),
            pl.BlockSpec((D, D), lambda i: (0, 0)),
        ],
        out_specs=pl.BlockSpec((BM, D), lambda i: (i, 0)),
        out_shape=jax.ShapeDtypeStruct((N, D), jnp.float32),
    )(sa, hp, dis_col, b_row, w)


LP = N // D


def _pool_body(sa_ref, hp_ref, dis_ref, b_ref, o_ref):
    h = dis_ref[...] * (sa_ref[...] + hp_ref[...]) + b_ref[...]
    o_ref[...] = jnp.mean(h, axis=0, keepdims=True)[None]


def _pool_call(sa, hp, dis_col, b_row):
    return pl.pallas_call(
        _pool_body,
        grid=(LP,),
        in_specs=[
            pl.BlockSpec((D, D), lambda i: (i, 0)),
            pl.BlockSpec((D, D), lambda i: (i, 0)),
            pl.BlockSpec((D, 1), lambda i: (i, 0)),
            pl.BlockSpec((1, D), lambda i: (0, 0)),
        ],
        out_specs=pl.BlockSpec((1, 1, D), lambda i: (i, 0, 0)),
        out_shape=jax.ShapeDtypeStruct((LP, 1, D), jnp.float32),
    )(sa, hp, dis_col, b_row).reshape(LP, D)


def _fc_body(p1_ref, p2_ref, w1_ref, w2_ref, b_ref, o_ref):
    dn = (((0,), (0,)), ((), ()))
    a = lax.dot_general(p1_ref[...], w1_ref[...], dn,
                        preferred_element_type=jnp.float32)
    a += lax.dot_general(p2_ref[...], w2_ref[...], dn,
                         preferred_element_type=jnp.float32)
    o_ref[...] = jax.nn.sigmoid(a + b_ref[...])


def _fc_call(p1, p2, w1, w2, b):
    return pl.pallas_call(
        _fc_body,
        out_shape=jax.ShapeDtypeStruct((D, 1), jnp.float32),
    )(p1, p2, w1, w2, b)


def _pad_edges(edge_index, edge_weight):
    src = edge_index[0].astype(jnp.int32)
    dst = edge_index[1].astype(jnp.int32)
    w = edge_weight
    e = src.shape[0]
    align = NW * CD
    e_pad = -(-e // align) * align
    if e_pad != e:
        pad = e_pad - e
        src = jnp.pad(src, (0, pad))
        dst = jnp.pad(dst, (0, pad))
        w = jnp.pad(w, (0, pad))
    return src, dst, w, e_pad


def kernel(x1, edge_index1, edge_weight1, x2, edge_index2, edge_weight2,
           W1a, b1a, W1b, b1b, W2a, b2a, W2b, b2b, Wfc, bfc):
    src1, dst1, w1, e1p = _pad_edges(edge_index1, edge_weight1)
    src2, dst2, w2, e2p = _pad_edges(edge_index2, edge_weight2)

    deg1 = _make_deg(e1p)
    deg2 = _make_deg(e1p) if e2p == e1p else _make_deg(e2p)
    spmm1 = _make_spmm(e1p)
    spmm2 = spmm1 if e2p == e1p else _make_spmm(e2p)

    parts1 = deg1(dst1, w1)
    parts2 = deg2(dst2, w2)
    dis1_row, dis2_row = _dis_call(parts1, parts2)
    dis1 = dis1_row.reshape(N, 1)
    dis2 = dis2_row.reshape(N, 1)

    h1p = _mm_call(x1, W1a, dis1)
    s1 = spmm1(h1p, src1, dst1, w1)
    h1q = _finmm_call(s1, h1p, dis1, b1a.reshape(1, D), W1b)
    s1b = spmm1(h1q, src1, dst1, w1)
    p1 = _pool_call(s1b, h1q, dis1, b1b.reshape(1, D))

    h2p = _mm_call(x2, W2a, dis2)
    s2 = spmm2(h2p, src2, dst2, w2)
    h2q = _finmm_call(s2, h2p, dis2, b2a.reshape(1, D), W2b)
    s2b = spmm2(h2q, src2, dst2, w2)
    p2 = _pool_call(s2b, h2q, dis2, b2b.reshape(1, D))

    return _fc_call(p1, p2, Wfc[:LP], Wfc[LP:], bfc.reshape(1, 1))


# speedup vs baseline: 1.1049x; 1.1049x over previous
import jax
import jax.numpy as jnp
from jax import lax
from jax.experimental import pallas as pl
from jax.experimental.pallas import tpu as pltpu
from jax.experimental.pallas import tpu_sc as plsc

N = 10000
D = 128
NC = 2
NS = 16
LN = 16
NW = NC * NS
C = 128
NP = 10240
RPT = NP // NS
ZB = 64
CD = 1024

SC0_FRAC = 0.75


def _sc_mesh():
    return plsc.VectorSubcoreMesh(
        core_axis_name="c", subcore_axis_name="s", num_cores=NC, num_subcores=NS
    )


def _make_deg(e_pad):
    per_tile = e_pad // NW
    nchunks = per_tile // CD

    def body(dst_hbm, w_hbm, out_hbm, dbuf, wbuf, acc):
        c = lax.axis_index("c")
        s = lax.axis_index("s")
        wid = c * NS + s

        zero = jnp.zeros((LN,), jnp.float32)

        def zloop(i, _):
            acc[pl.ds(i * LN, LN)] = zero
            return 0

        lax.fori_loop(0, N // LN, zloop, 0)

        base = wid * per_tile

        def chunk(k, _):
            off = base + k * CD
            pltpu.sync_copy(dst_hbm.at[pl.ds(off, CD)], dbuf)
            pltpu.sync_copy(w_hbm.at[pl.ds(off, CD)], wbuf)

            def group(t, _):
                iv = dbuf[pl.ds(t * LN, LN)]
                wv = wbuf[pl.ds(t * LN, LN)]
                plsc.addupdate_scatter(acc, [iv], wv)
                return 0

            lax.fori_loop(0, CD // LN, group, 0)
            return 0

        lax.fori_loop(0, nchunks, chunk, 0)
        pltpu.sync_copy(acc, out_hbm.at[wid])

    return pl.kernel(
        body,
        out_type=jax.ShapeDtypeStruct((NW, N), jnp.float32),
        mesh=_sc_mesh(),
        compiler_params=pltpu.CompilerParams(needs_layout_passes=False),
        scratch_types=[
            pltpu.VMEM((CD,), jnp.int32),
            pltpu.VMEM((CD,), jnp.float32),
            pltpu.VMEM((N,), jnp.float32),
        ],
    )


def _make_spmm(e_pad):
    nchunks = e_pad // C
    totq = nchunks // (4 * NS)
    q0 = max(2, min(totq - 2, round(totq * SC0_FRAC)))
    q1 = totq - q0
    assert nchunks % (4 * NS) == 0 and q1 >= 2

    def body(h_hbm, src_hbm, dst_hbm, w_hbm, out_hbm,
             s0, s1, s2, s3, d0, d1, d2, d3, w0, w1, w2, w3,
             r0, r1, zbuf, acc,
             si0, si1, si2, si3, sg0, sg1, ss0, ss1):
        c = lax.axis_index("c")
        s = lax.axis_index("s")

        zero = jnp.zeros((LN,), jnp.float32)

        def zloop(i, _):
            zbuf[i // (D // LN), pl.ds((i % (D // LN)) * LN, LN)] = zero
            return 0

        lax.fori_loop(0, ZB * (D // LN), zloop, 0)

        rbase = s * RPT

        def zcopy(i, _):
            pltpu.sync_copy(zbuf, acc.at[pl.ds(rbase + i * ZB, ZB)])
            return 0

        lax.fori_loop(0, RPT // ZB, zcopy, 0)
        plsc.subcore_barrier()

        nq = jnp.where(c == 0, q0, q1)
        chunk0 = jnp.where(c == 0, s * (4 * q0), NS * (4 * q0) + s * (4 * q1))
        base = chunk0 * C
        cmax = 4 * nq - 1
        SB = [s0, s1, s2, s3]
        DB = [d0, d1, d2, d3]
        WB = [w0, w1, w2, w3]
        SI = [si0, si1, si2, si3]

        def idx_start(ck, j):
            off = base + jnp.minimum(ck, cmax) * C
            return (pltpu.async_copy(src_hbm.at[pl.ds(off, C)], SB[j], SI[j]),
                    pltpu.async_copy(dst_hbm.at[pl.ds(off, C)], DB[j], SI[j]),
                    pltpu.async_copy(w_hbm.at[pl.ds(off, C)], WB[j], SI[j]))

        def idx_wait(j):
            pltpu.make_async_copy(src_hbm.at[pl.ds(base, C)], SB[j], SI[j]).wait()
            pltpu.make_async_copy(dst_hbm.at[pl.ds(base, C)], DB[j], SI[j]).wait()
            pltpu.make_async_copy(w_hbm.at[pl.ds(base, C)], WB[j], SI[j]).wait()

        def gather_start(j, r, sem):
            return pltpu.async_copy(h_hbm.at[SB[j]], r, sem)

        def gather_wait(j, r, sem):
            pltpu.make_async_copy(h_hbm.at[SB[j]], r, sem).wait()

        def scale(r, wref):
            def sc16(t, _):
                wv = wref[pl.ds(t * LN, LN)]
                for l in range(LN):
                    wsv = jnp.full((LN,), wv[l], jnp.float32)
                    e = t * LN + l
                    for j in range(D // LN):
                        r[e, pl.ds(j * LN, LN)] = r[e, pl.ds(j * LN, LN)] * wsv
                return 0

            lax.fori_loop(0, C // LN, sc16, 0)

        pro = [idx_start(j, j) for j in range(4)]
        for t in pro[0]:
            t.wait()
        gather_start(0, r0, sg0)
        for t in pro[1]:
            t.wait()
        gather_start(1, r1, sg1)

        def quad(q, _):
            c0 = 4 * q
            gather_wait(0, r0, sg0)
            scale(r0, w0)
            sc_a = pltpu.async_copy(r0, acc.at[d0], ss0, add=True)
            gather_wait(1, r1, sg1)
            scale(r1, w1)
            sc_b = pltpu.async_copy(r1, acc.at[d1], ss1, add=True)
            sc_a.wait()
            idx_wait(2)
            gather_start(2, r0, sg0)
            idx_start(c0 + 4, 0)
            sc_b.wait()
            idx_wait(3)
            gather_start(3, r1, sg1)
            idx_start(c0 + 5, 1)
            gather_wait(2, r0, sg0)
            scale(r0, w2)
            sc_c = pltpu.async_copy(r0, acc.at[d2], ss0, add=True)
            gather_wait(3, r1, sg1)
            scale(r1, w3)
            sc_d = pltpu.async_copy(r1, acc.at[d3], ss1, add=True)
            sc_c.wait()
            idx_wait(0)
            gather_start(0, r0, sg0)
            idx_start(c0 + 6, 2)
            sc_d.wait()
            idx_wait(1)
            gather_start(1, r1, sg1)
            idx_start(c0 + 7, 3)
            return 0

        lax.fori_loop(0, nq, quad, 0)

        gather_wait(0, r0, sg0)
        gather_wait(1, r1, sg1)
        idx_wait(2)
        idx_wait(3)

        plsc.subcore_barrier()

        def wback(i, _):
            pltpu.sync_copy(acc.at[pl.ds(rbase + i * ZB, ZB)],
                            out_hbm.at[c, pl.ds(rbase + i * ZB, ZB)])
            return 0

        lax.fori_loop(0, RPT // ZB, wback, 0)

    return pl.kernel(
        body,
        out_type=jax.ShapeDtypeStruct((NC, NP, D), jnp.float32),
        mesh=_sc_mesh(),
        compiler_params=pltpu.CompilerParams(needs_layout_passes=False),
        scratch_types=(
            [pltpu.VMEM((C,), jnp.int32)] * 8
            + [pltpu.VMEM((C,), jnp.float32)] * 4
            + [pltpu.VMEM((C, D), jnp.float32)] * 2
            + [pltpu.VMEM((ZB, D), jnp.float32),
               pltpu.VMEM_SHARED((NP, D), jnp.float32)]
            + [pltpu.SemaphoreType.DMA] * 8
        ),
    )


def _dis_body(p1_ref, p2_ref, o1_ref, o2_ref):
    o1_ref[...] = lax.rsqrt(1.0 + jnp.sum(p1_ref[...], axis=0, keepdims=True))
    o2_ref[...] = lax.rsqrt(1.0 + jnp.sum(p2_ref[...], axis=0, keepdims=True))


def _dis_call(p1, p2):
    return pl.pallas_call(
        _dis_body,
        out_shape=(jax.ShapeDtypeStruct((1, N), jnp.float32),
                   jax.ShapeDtypeStruct((1, N), jnp.float32)),
    )(p1, p2)


BM = 1000


def _mm_body(x_ref, w_ref, dis_ref, o_ref):
    h = jnp.dot(x_ref[...], w_ref[...], preferred_element_type=jnp.float32)
    o_ref[...] = h * dis_ref[...]


def _mm_call(x, w, dis_col):
    return pl.pallas_call(
        _mm_body,
        grid=(N // BM,),
        in_specs=[
            pl.BlockSpec((BM, D), lambda i: (i, 0)),
            pl.BlockSpec((D, D), lambda i: (0, 0)),
            pl.BlockSpec((BM, 1), lambda i: (i, 0)),
        ],
        out_specs=pl.BlockSpec((BM, D), lambda i: (i, 0)),
        out_shape=jax.ShapeDtypeStruct((N, D), jnp.float32),
    )(x, w, dis_col)


def _finmm_body(sa_ref, sb_ref, hp_ref, dis_ref, b_ref, w_ref, o_ref):
    h = dis_ref[...] * (sa_ref[...] + sb_ref[...] + hp_ref[...]) + b_ref[...]
    o_ref[...] = dis_ref[...] * jnp.dot(h, w_ref[...],
                                        preferred_element_type=jnp.float32)


def _finmm_call(sa, sb, hp, dis_col, b_row, w):
    return pl.pallas_call(
        _finmm_body,
        grid=(N // BM,),
        in_specs=[
            pl.BlockSpec((BM, D), lambda i: (i, 0)),
            pl.BlockSpec((BM, D), lambda i: (i, 0)),
            pl.BlockSpec((BM, D), lambda i: (i, 0)),
            pl.BlockSpec((BM, 1), lambda i: (i, 0)),
            pl.BlockSpec((1, D), lambda i: (0, 0)),
            pl.BlockSpec((D, D), lambda i: (0, 0)),
        ],
        out_specs=pl.BlockSpec((BM, D), lambda i: (i, 0)),
        out_shape=jax.ShapeDtypeStruct((N, D), jnp.float32),
    )(sa, sb, hp, dis_col, b_row, w)


LP = N // D


def _pool_body(sa_ref, sb_ref, hp_ref, dis_ref, b_ref, o_ref):
    sp = sa_ref[...] + sb_ref[...]
    h = dis_ref[...] * (sp + hp_ref[...]) + b_ref[...]
    o_ref[...] = jnp.mean(h, axis=0, keepdims=True)[None]


def _pool_call(sa, sb, hp, dis_col, b_row):
    return pl.pallas_call(
        _pool_body,
        grid=(LP,),
        in_specs=[
            pl.BlockSpec((D, D), lambda i: (i, 0)),
            pl.BlockSpec((D, D), lambda i: (i, 0)),
            pl.BlockSpec((D, D), lambda i: (i, 0)),
            pl.BlockSpec((D, 1), lambda i: (i, 0)),
            pl.BlockSpec((1, D), lambda i: (0, 0)),
        ],
        out_specs=pl.BlockSpec((1, 1, D), lambda i: (i, 0, 0)),
        out_shape=jax.ShapeDtypeStruct((LP, 1, D), jnp.float32),
    )(sa, sb, hp, dis_col, b_row).reshape(LP, D)


def _fc_body(p1_ref, p2_ref, w1_ref, w2_ref, b_ref, o_ref):
    dn = (((0,), (0,)), ((), ()))
    a = lax.dot_general(p1_ref[...], w1_ref[...], dn,
                        preferred_element_type=jnp.float32)
    a += lax.dot_general(p2_ref[...], w2_ref[...], dn,
                         preferred_element_type=jnp.float32)
    o_ref[...] = jax.nn.sigmoid(a + b_ref[...])


def _fc_call(p1, p2, w1, w2, b):
    return pl.pallas_call(
        _fc_body,
        out_shape=jax.ShapeDtypeStruct((D, 1), jnp.float32),
    )(p1, p2, w1, w2, b)


def _pad_edges(edge_index, edge_weight):
    src = edge_index[0].astype(jnp.int32)
    dst = edge_index[1].astype(jnp.int32)
    w = edge_weight
    e = src.shape[0]
    align = NW * CD
    e_pad = -(-e // align) * align
    if e_pad != e:
        pad = e_pad - e
        src = jnp.pad(src, (0, pad))
        dst = jnp.pad(dst, (0, pad))
        w = jnp.pad(w, (0, pad))
    return src, dst, w, e_pad


def kernel(x1, edge_index1, edge_weight1, x2, edge_index2, edge_weight2,
           W1a, b1a, W1b, b1b, W2a, b2a, W2b, b2b, Wfc, bfc):
    src1, dst1, w1, e1p = _pad_edges(edge_index1, edge_weight1)
    src2, dst2, w2, e2p = _pad_edges(edge_index2, edge_weight2)

    deg1 = _make_deg(e1p)
    deg2 = _make_deg(e1p) if e2p == e1p else _make_deg(e2p)
    spmm1 = _make_spmm(e1p)
    spmm2 = spmm1 if e2p == e1p else _make_spmm(e2p)

    parts1 = deg1(dst1, w1)
    parts2 = deg2(dst2, w2)
    dis1_row, dis2_row = _dis_call(parts1, parts2)
    dis1 = dis1_row.reshape(N, 1)
    dis2 = dis2_row.reshape(N, 1)

    h1p = _mm_call(x1, W1a, dis1)
    s1 = spmm1(h1p, src1, dst1, w1)
    h1q = _finmm_call(s1[0], s1[1], h1p, dis1, b1a.reshape(1, D), W1b)
    s1b = spmm1(h1q, src1, dst1, w1)
    p1 = _pool_call(s1b[0], s1b[1], h1q, dis1, b1b.reshape(1, D))

    h2p = _mm_call(x2, W2a, dis2)
    s2 = spmm2(h2p, src2, dst2, w2)
    h2q = _finmm_call(s2[0], s2[1], h2p, dis2, b2a.reshape(1, D), W2b)
    s2b = spmm2(h2q, src2, dst2, w2)
    p2 = _pool_call(s2b[0], s2b[1], h2q, dis2, b2b.reshape(1, D))

    return _fc_call(p1, p2, Wfc[:LP], Wfc[LP:], bfc.reshape(1, 1))
